# baseline (speedup 1.0000x reference)
import jax
import jax.numpy as jnp
from jax import lax
from jax.experimental import pallas as pl
from jax.experimental.pallas import tpu as pltpu
from jax.experimental.pallas import tpu_sc as plsc

F = 26
V = 100000
D = 32
B = 16384

_info = plsc.get_sparse_core_info()
NC = _info.num_cores
NSUB = _info.num_subcores
L = _info.num_lanes
NW = NC * NSUB
RW = B // NW
SUB = 128
NCH = RW // SUB


def _body(idx_hbm, tab_hbm, out_hbm, idxraw, idxbuf, buf0, buf1, acc, sem0, sem1):
    c = lax.axis_index("c")
    s = lax.axis_index("s")
    wid = c * NSUB + s
    base = wid * NCH

    pltpu.sync_copy(idx_hbm.at[pl.ds(wid * RW, RW), :], idxraw)

    iota = lax.iota(jnp.int32, L)

    def _off_field(i, _):
        off = i * V
        col = jnp.full((L,), 0, jnp.int32) + i
        for k in range(NCH):
            for j in range(SUB // L):
                rows = iota + (k * SUB + j * L)
                v = plsc.load_gather(idxraw, [rows, col])
                idxbuf[i, k, pl.ds(j * L, L)] = v + off
        return 0

    lax.fori_loop(0, F, _off_field, 0)

    bufs = (buf0, buf1)
    sems = (sem0, sem1)

    def _fire(i, k, p):
        pltpu.async_copy(tab_hbm.at[idxbuf.at[i, k]], bufs[p], sems[p])

    def _wait(p):
        pltpu.make_async_copy(tab_hbm.at[idxbuf.at[0, 0]], bufs[p], sems[p]).wait()

    _fire(0, 0, 0)

    def _chunk(k, _):
        for i in range(F):
            p = i & 1
            q = (i + 1) & 1
            if i < F - 1:
                _fire(i + 1, k, q)
            else:
                @pl.when(k < NCH - 1)
                def _next_chunk_fire():
                    _fire(0, k + 1, q)
            _wait(p)
            bp = bufs[p]
            if i == 0:
                def _cp(r8, _c):
                    for rr in range(8):
                        r = r8 * 8 + rr
                        for h in range(D // L):
                            acc[r, pl.ds(h * L, L)] = bp[r, pl.ds(h * L, L)]
                    return 0
                lax.fori_loop(0, SUB // 8, _cp, 0)
            else:
                def _ad(r8, _c):
                    for rr in range(8):
                        r = r8 * 8 + rr
                        for h in range(D // L):
                            plsc.addupdate(acc.at[r, pl.ds(h * L, L)],
                                           bp[r, pl.ds(h * L, L)])
                    return 0
                lax.fori_loop(0, SUB // 8, _ad, 0)
        pltpu.sync_copy(acc, out_hbm.at[pl.ds((base + k) * SUB, SUB), :])
        return 0

    lax.fori_loop(0, NCH, _chunk, 0)


def kernel(inputs, tables):
    idx = inputs.astype(jnp.int32)
    tab_flat = tables.reshape(F * V, D)
    mesh = plsc.VectorSubcoreMesh(core_axis_name="c", subcore_axis_name="s")
    f = pl.kernel(
        _body,
        out_type=jax.ShapeDtypeStruct((B, D), jnp.float32),
        mesh=mesh,
        scratch_types=[
            pltpu.VMEM((RW, F), jnp.int32),
            pltpu.VMEM((F, NCH, SUB), jnp.int32),
            pltpu.VMEM((SUB, D), jnp.float32),
            pltpu.VMEM((SUB, D), jnp.float32),
            pltpu.VMEM((SUB, D), jnp.float32),
            pltpu.SemaphoreType.DMA,
            pltpu.SemaphoreType.DMA,
        ],
        compiler_params=pltpu.CompilerParams(use_tc_tiling_on_sc=False,
                                             needs_layout_passes=False),
    )
    return f(idx, tab_flat)


# speedup vs baseline: 4.1033x; 4.1033x over previous
import jax
import jax.numpy as jnp
from jax import lax
from jax.experimental import pallas as pl
from jax.experimental.pallas import tpu as pltpu
from jax.experimental.pallas import tpu_sc as plsc

F = 26
V = 100000
D = 32
B = 16384

_info = plsc.get_sparse_core_info()
NC = _info.num_cores
NSUB = _info.num_subcores
L = _info.num_lanes
NW = NC * NSUB
BC = 8192
NBC = B // BC
UNROLL = 8


def _body(idx_hbm, tab_hbm, out_hbm, rowbuf, idxbuf, acc):
    c = lax.axis_index("c")
    s = lax.axis_index("s")
    d = c * NSUB + s

    def _field(i, _):
        pltpu.sync_copy(tab_hbm.at[i, d], rowbuf)

        def _chunk(h, _c):
            pltpu.sync_copy(idx_hbm.at[i, pl.ds(h * BC, BC)], idxbuf)
            base = h * BC

            def _gat_first(j, _g):
                for u in range(UNROLL):
                    off = (j * UNROLL + u) * L
                    v = idxbuf[pl.ds(off, L)]
                    vals = plsc.load_gather(rowbuf, [v])
                    acc[pl.ds(base + off, L)] = vals
                return 0

            def _gat_add(j, _g):
                for u in range(UNROLL):
                    off = (j * UNROLL + u) * L
                    v = idxbuf[pl.ds(off, L)]
                    vals = plsc.load_gather(rowbuf, [v])
                    plsc.addupdate(acc.at[pl.ds(base + off, L)], vals)
                return 0

            nsl = BC // (L * UNROLL)
            lax.cond(i == 0,
                     lambda: lax.fori_loop(0, nsl, _gat_first, 0),
                     lambda: lax.fori_loop(0, nsl, _gat_add, 0))
            return 0

        lax.fori_loop(0, NBC, _chunk, 0)
        return 0

    lax.fori_loop(0, F, _field, 0)
    pltpu.sync_copy(acc, out_hbm.at[d])


def kernel(inputs, tables):
    idx_t = jnp.transpose(inputs).astype(jnp.int32)
    tab_t = jnp.transpose(tables, (0, 2, 1))
    mesh = plsc.VectorSubcoreMesh(core_axis_name="c", subcore_axis_name="s")
    f = pl.kernel(
        _body,
        out_type=jax.ShapeDtypeStruct((D, B), jnp.float32),
        mesh=mesh,
        scratch_types=[
            pltpu.VMEM((V,), jnp.float32),
            pltpu.VMEM((BC,), jnp.int32),
            pltpu.VMEM((B,), jnp.float32),
        ],
        compiler_params=pltpu.CompilerParams(needs_layout_passes=False),
    )
    out_t = f(idx_t, tab_t)
    return jnp.transpose(out_t)
